# baseline (speedup 1.0000x reference)
import jax
import jax.numpy as jnp
from jax import lax
from jax.experimental import pallas as pl
from jax.experimental.pallas import tpu as pltpu
from jax.experimental.pallas import tpu_sc as plsc

_EPS = 1e-5

_S = 8192
_B = 4
_D = 1024
_N = float(_S * _B)

_S_TC = 5120
_S_SC = _S - _S_TC
_NW = 32
_SPAN = _S_SC // _NW
_CH = 8
_NCHUNK = _SPAN // _CH
_NG = _D // 16

_SB = 256
_SBN = 512


def _wid():
    return lax.axis_index("s") * 2 + lax.axis_index("c")


def _sc_stats_body(x_hbm, pe_hbm, part_hbm,
                   xbuf0, xbuf1, pebuf0, pebuf1, acc,
                   semx0, semx1, semp0, semp1):
    base = _S_TC + _wid() * _SPAN
    xbufs = (xbuf0, xbuf1)
    pebufs = (pebuf0, pebuf1)
    semxs = (semx0, semx1)
    semps = (semp0, semp1)
    zero = jnp.zeros((16,), jnp.float32)

    def zbody(i, _):
        acc[0, pl.ds(i * 16, 16)] = zero
        acc[1, pl.ds(i * 16, 16)] = zero
        return 0

    lax.fori_loop(0, _NG, zbody, 0)

    def start_in(ci, slot):
        row = base + ci * _CH
        pltpu.async_copy(x_hbm.at[pl.ds(row, _CH)], xbufs[slot], semxs[slot])
        pltpu.async_copy(pe_hbm.at[pl.ds(row, _CH)], pebufs[slot], semps[slot])

    def wait_in(ci, slot):
        row = base + ci * _CH
        pltpu.make_async_copy(
            x_hbm.at[pl.ds(row, _CH)], xbufs[slot], semxs[slot]).wait()
        pltpu.make_async_copy(
            pe_hbm.at[pl.ds(row, _CH)], pebufs[slot], semps[slot]).wait()

    start_in(0, 0)
    for ci in range(_NCHUNK):
        slot = ci % 2
        if ci + 1 < _NCHUNK:
            start_in(ci + 1, (ci + 1) % 2)
        wait_in(ci, slot)
        xbuf = xbufs[slot]
        pebuf = pebufs[slot]

        @plsc.parallel_loop(0, _NG, 1, unroll=4)
        def gbody(g, xbuf=xbuf, pebuf=pebuf):
            off = g * 16
            sv = zero
            qv = zero
            for r in range(_CH):
                pv = pebuf[r, pl.ds(off, 16)]
                for b in range(_B):
                    y = xbuf[r, b, pl.ds(off, 16)] + pv
                    sv = sv + y
                    qv = qv + y * y
            acc[0, pl.ds(off, 16)] += sv
            acc[1, pl.ds(off, 16)] += qv

    pltpu.sync_copy(acc, part_hbm.at[_wid()])


def _tc_stats_body(x_ref, pe_ref, sums_ref):
    i = pl.program_id(0)
    y = x_ref[...] + pe_ref[...][:, None, :]
    s = jnp.sum(y, axis=0)
    q = jnp.sum(y * y, axis=0)
    part = jnp.stack([s, q])

    @pl.when(i == 0)
    def _():
        sums_ref[...] = part

    @pl.when(i != 0)
    def _():
        sums_ref[...] += part


def _scale_body(part_ref, sums_ref, gamma_ref, beta_ref, go_ref):
    sums = (jnp.sum(part_ref[...], axis=0)
            + jnp.sum(sums_ref[...], axis=1))
    mean = sums[0:1, :] / _N
    var = sums[1:2, :] / _N - mean * mean
    inv = lax.rsqrt(var + _EPS)
    g = gamma_ref[...] * inv
    off = beta_ref[...] - mean * g
    go_ref[...] = jnp.concatenate([g, off], axis=0)


def _tc_norm_body(go_ref, x_ref, pe_ref, o_ref):
    g = go_ref[0:1, :]
    off = go_ref[1:2, :]
    row = pe_ref[...] * g + off
    o_ref[...] = x_ref[...] * g[:, None, :] + row[:, None, :]


def kernel(x, pos_table, gamma, beta):
    S, B, D = x.shape
    mesh = plsc.VectorSubcoreMesh(core_axis_name="c", subcore_axis_name="s")
    pe = pos_table[:S]

    sc_stats = pl.kernel(
        _sc_stats_body,
        out_type=jax.ShapeDtypeStruct((_NW, 2, D), jnp.float32),
        mesh=mesh,
        scratch_types=[
            pltpu.VMEM((_CH, B, D), jnp.float32),
            pltpu.VMEM((_CH, B, D), jnp.float32),
            pltpu.VMEM((_CH, D), jnp.float32),
            pltpu.VMEM((_CH, D), jnp.float32),
            pltpu.VMEM((2, D), jnp.float32),
            pltpu.SemaphoreType.DMA,
            pltpu.SemaphoreType.DMA,
            pltpu.SemaphoreType.DMA,
            pltpu.SemaphoreType.DMA,
        ],
    )
    partials = sc_stats(x, pe)

    sums_tc = pl.pallas_call(
        _tc_stats_body,
        grid=(_S_TC // _SB,),
        in_specs=[
            pl.BlockSpec((_SB, B, D), lambda i: (i, 0, 0)),
            pl.BlockSpec((_SB, D), lambda i: (i, 0)),
        ],
        out_specs=pl.BlockSpec((2, B, D), lambda i: (0, 0, 0)),
        out_shape=jax.ShapeDtypeStruct((2, B, D), jnp.float32),
    )(x, pe)

    go = pl.pallas_call(
        _scale_body,
        out_shape=jax.ShapeDtypeStruct((2, D), jnp.float32),
    )(partials, sums_tc, gamma.reshape(1, D), beta.reshape(1, D))

    out = pl.pallas_call(
        _tc_norm_body,
        grid=(S // _SBN,),
        in_specs=[
            pl.BlockSpec((2, D), lambda i: (0, 0)),
            pl.BlockSpec((_SBN, B, D), lambda i: (i, 0, 0)),
            pl.BlockSpec((_SBN, D), lambda i: (i, 0)),
        ],
        out_specs=pl.BlockSpec((_SBN, B, D), lambda i: (i, 0, 0)),
        out_shape=jax.ShapeDtypeStruct((S, B, D), jnp.float32),
    )(go, x, pe)

    return out


# speedup vs baseline: 1.0374x; 1.0374x over previous
import jax
import jax.numpy as jnp
from jax import lax
from jax.experimental import pallas as pl
from jax.experimental.pallas import tpu as pltpu
from jax.experimental.pallas import tpu_sc as plsc

_EPS = 1e-5

_S = 8192
_B = 4
_D = 1024
_N = float(_S * _B)

_S_TC = 5120
_S_SC = _S - _S_TC
_NW = 32
_SPAN = _S_SC // _NW
_CH = 8
_NCHUNK = _SPAN // _CH
_NG = _D // 16

_SB = 256
_SBN = 512


def _wid():
    return lax.axis_index("s") * 2 + lax.axis_index("c")


def _sc_stats_body(x_hbm, pe_hbm, part_hbm,
                   xbuf0, xbuf1, pebuf0, pebuf1, acc,
                   semx0, semx1, semp0, semp1):
    base = _S_TC + _wid() * _SPAN
    xbufs = (xbuf0, xbuf1)
    pebufs = (pebuf0, pebuf1)
    semxs = (semx0, semx1)
    semps = (semp0, semp1)
    zero = jnp.zeros((16,), jnp.float32)

    def zbody(i, _):
        acc[0, pl.ds(i * 16, 16)] = zero
        acc[1, pl.ds(i * 16, 16)] = zero
        return 0

    lax.fori_loop(0, _NG, zbody, 0)

    def start_in(ci, slot):
        row = base + ci * _CH
        pltpu.async_copy(x_hbm.at[pl.ds(row, _CH)], xbufs[slot], semxs[slot])
        pltpu.async_copy(pe_hbm.at[pl.ds(row, _CH)], pebufs[slot], semps[slot])

    def wait_in(ci, slot):
        row = base + ci * _CH
        pltpu.make_async_copy(
            x_hbm.at[pl.ds(row, _CH)], xbufs[slot], semxs[slot]).wait()
        pltpu.make_async_copy(
            pe_hbm.at[pl.ds(row, _CH)], pebufs[slot], semps[slot]).wait()

    start_in(0, 0)
    for ci in range(_NCHUNK):
        slot = ci % 2
        if ci + 1 < _NCHUNK:
            start_in(ci + 1, (ci + 1) % 2)
        wait_in(ci, slot)
        xbuf = xbufs[slot]
        pebuf = pebufs[slot]

        @plsc.parallel_loop(0, _NG, 1, unroll=4)
        def gbody(g, xbuf=xbuf, pebuf=pebuf):
            off = g * 16
            sv = zero
            qv = zero
            for r in range(_CH):
                pv = pebuf[r, pl.ds(off, 16)]
                for b in range(_B):
                    y = xbuf[r, b, pl.ds(off, 16)] + pv
                    sv = sv + y
                    qv = qv + y * y
            acc[0, pl.ds(off, 16)] += sv
            acc[1, pl.ds(off, 16)] += qv

    pltpu.sync_copy(acc, part_hbm.at[_wid()])


_RCH = 8


def _tc_stats_body(x_ref, pe_ref, sums_ref):
    i = pl.program_id(0)

    def body(k, carry):
        s, q = carry
        y = x_ref[pl.ds(k * _RCH, _RCH)] + pe_ref[pl.ds(k * _RCH, _RCH)][:, None, :]
        s = s + jnp.sum(y, axis=0)
        q = q + jnp.sum(y * y, axis=0)
        return (s, q)

    z = jnp.zeros((_B, _D), jnp.float32)
    s, q = lax.fori_loop(0, _SB // _RCH, body, (z, z))
    part = jnp.stack([s, q])

    @pl.when(i == 0)
    def _():
        sums_ref[...] = part

    @pl.when(i != 0)
    def _():
        sums_ref[...] += part


def _scale_body(part_ref, sums_ref, gamma_ref, beta_ref, go_ref):
    sums = (jnp.sum(part_ref[...], axis=0)
            + jnp.sum(sums_ref[...], axis=1))
    mean = sums[0:1, :] / _N
    var = sums[1:2, :] / _N - mean * mean
    inv = lax.rsqrt(var + _EPS)
    g = gamma_ref[...] * inv
    off = beta_ref[...] - mean * g
    go_ref[...] = jnp.concatenate([g, off], axis=0)


def _tc_norm_body(go_ref, x_ref, pe_ref, o_ref):
    g = go_ref[0:1, :]
    off = go_ref[1:2, :]
    row = pe_ref[...] * g + off
    o_ref[...] = x_ref[...] * g[:, None, :] + row[:, None, :]


def kernel(x, pos_table, gamma, beta):
    S, B, D = x.shape
    mesh = plsc.VectorSubcoreMesh(core_axis_name="c", subcore_axis_name="s")
    pe = pos_table[:S]

    sc_stats = pl.kernel(
        _sc_stats_body,
        out_type=jax.ShapeDtypeStruct((_NW, 2, D), jnp.float32),
        mesh=mesh,
        scratch_types=[
            pltpu.VMEM((_CH, B, D), jnp.float32),
            pltpu.VMEM((_CH, B, D), jnp.float32),
            pltpu.VMEM((_CH, D), jnp.float32),
            pltpu.VMEM((_CH, D), jnp.float32),
            pltpu.VMEM((2, D), jnp.float32),
            pltpu.SemaphoreType.DMA,
            pltpu.SemaphoreType.DMA,
            pltpu.SemaphoreType.DMA,
            pltpu.SemaphoreType.DMA,
        ],
    )
    partials = sc_stats(x, pe)

    sums_tc = pl.pallas_call(
        _tc_stats_body,
        grid=(_S_TC // _SB,),
        in_specs=[
            pl.BlockSpec((_SB, B, D), lambda i: (i, 0, 0)),
            pl.BlockSpec((_SB, D), lambda i: (i, 0)),
        ],
        out_specs=pl.BlockSpec((2, B, D), lambda i: (0, 0, 0)),
        out_shape=jax.ShapeDtypeStruct((2, B, D), jnp.float32),
    )(x, pe)

    go = pl.pallas_call(
        _scale_body,
        out_shape=jax.ShapeDtypeStruct((2, D), jnp.float32),
    )(partials, sums_tc, gamma.reshape(1, D), beta.reshape(1, D))

    out = pl.pallas_call(
        _tc_norm_body,
        grid=(S // _SBN,),
        in_specs=[
            pl.BlockSpec((2, D), lambda i: (0, 0)),
            pl.BlockSpec((_SBN, B, D), lambda i: (i, 0, 0)),
            pl.BlockSpec((_SBN, D), lambda i: (i, 0)),
        ],
        out_specs=pl.BlockSpec((_SBN, B, D), lambda i: (i, 0, 0)),
        out_shape=jax.ShapeDtypeStruct((S, B, D), jnp.float32),
    )(go, x, pe)

    return out


# speedup vs baseline: 1.0800x; 1.0410x over previous
import jax
import jax.numpy as jnp
from jax import lax
from jax.experimental import pallas as pl
from jax.experimental.pallas import tpu as pltpu
from jax.experimental.pallas import tpu_sc as plsc

_EPS = 1e-5

_S = 8192
_B = 4
_D = 1024
_N = float(_S * _B)

_S_TC = 4352
_S_SC = _S - _S_TC
_NW = 32
_SPAN = _S_SC // _NW
_CH = 8
_NCHUNK = _SPAN // _CH
_NG = _D // 16

_SB = 256
_SBN = 512


def _wid():
    return lax.axis_index("s") * 2 + lax.axis_index("c")


def _sc_stats_body(x_hbm, pe_hbm, part_hbm,
                   xbuf0, xbuf1, pebuf0, pebuf1, acc,
                   semx0, semx1, semp0, semp1):
    base = _S_TC + _wid() * _SPAN
    xbufs = (xbuf0, xbuf1)
    pebufs = (pebuf0, pebuf1)
    semxs = (semx0, semx1)
    semps = (semp0, semp1)
    zero = jnp.zeros((16,), jnp.float32)

    def zbody(i, _):
        acc[0, pl.ds(i * 16, 16)] = zero
        acc[1, pl.ds(i * 16, 16)] = zero
        return 0

    lax.fori_loop(0, _NG, zbody, 0)

    def start_in(ci, slot):
        row = base + ci * _CH
        pltpu.async_copy(x_hbm.at[pl.ds(row, _CH)], xbufs[slot], semxs[slot])
        pltpu.async_copy(pe_hbm.at[pl.ds(row, _CH)], pebufs[slot], semps[slot])

    def wait_in(ci, slot):
        row = base + ci * _CH
        pltpu.make_async_copy(
            x_hbm.at[pl.ds(row, _CH)], xbufs[slot], semxs[slot]).wait()
        pltpu.make_async_copy(
            pe_hbm.at[pl.ds(row, _CH)], pebufs[slot], semps[slot]).wait()

    start_in(0, 0)
    for ci in range(_NCHUNK):
        slot = ci % 2
        if ci + 1 < _NCHUNK:
            start_in(ci + 1, (ci + 1) % 2)
        wait_in(ci, slot)
        xbuf = xbufs[slot]
        pebuf = pebufs[slot]

        @plsc.parallel_loop(0, _NG, 1, unroll=4)
        def gbody(g, xbuf=xbuf, pebuf=pebuf):
            off = g * 16
            sv = zero
            qv = zero
            for r in range(_CH):
                pv = pebuf[r, pl.ds(off, 16)]
                for b in range(_B):
                    y = xbuf[r, b, pl.ds(off, 16)] + pv
                    sv = sv + y
                    qv = qv + y * y
            acc[0, pl.ds(off, 16)] += sv
            acc[1, pl.ds(off, 16)] += qv

    pltpu.sync_copy(acc, part_hbm.at[_wid()])


_RCH = 16


def _tc_stats_body(x_ref, pe_ref, sums_ref):
    i = pl.program_id(0)

    def body(k, carry):
        s0, q0, s1, q1 = carry
        base = k * _RCH
        y0 = x_ref[pl.ds(base, 8)] + pe_ref[pl.ds(base, 8)][:, None, :]
        y1 = x_ref[pl.ds(base + 8, 8)] + pe_ref[pl.ds(base + 8, 8)][:, None, :]
        s0 = s0 + jnp.sum(y0, axis=0)
        q0 = q0 + jnp.sum(y0 * y0, axis=0)
        s1 = s1 + jnp.sum(y1, axis=0)
        q1 = q1 + jnp.sum(y1 * y1, axis=0)
        return (s0, q0, s1, q1)

    z = jnp.zeros((_B, _D), jnp.float32)
    s0, q0, s1, q1 = lax.fori_loop(0, _SB // _RCH, body, (z, z, z, z))
    part = jnp.stack([s0 + s1, q0 + q1])

    @pl.when(i == 0)
    def _():
        sums_ref[...] = part

    @pl.when(i != 0)
    def _():
        sums_ref[...] += part


def _scale_body(part_ref, sums_ref, gamma_ref, beta_ref, go_ref):
    sums = (jnp.sum(part_ref[...], axis=0)
            + jnp.sum(sums_ref[...], axis=1))
    mean = sums[0:1, :] / _N
    var = sums[1:2, :] / _N - mean * mean
    inv = lax.rsqrt(var + _EPS)
    g = gamma_ref[...] * inv
    off = beta_ref[...] - mean * g
    go_ref[...] = jnp.concatenate([g, off], axis=0)


def _tc_norm_body(go_ref, x_ref, pe_ref, o_ref):
    g = go_ref[0:1, :]
    off = go_ref[1:2, :]
    row = pe_ref[...] * g + off
    o_ref[...] = x_ref[...] * g[:, None, :] + row[:, None, :]


def kernel(x, pos_table, gamma, beta):
    S, B, D = x.shape
    mesh = plsc.VectorSubcoreMesh(core_axis_name="c", subcore_axis_name="s")
    pe = pos_table[:S]

    sc_stats = pl.kernel(
        _sc_stats_body,
        out_type=jax.ShapeDtypeStruct((_NW, 2, D), jnp.float32),
        mesh=mesh,
        scratch_types=[
            pltpu.VMEM((_CH, B, D), jnp.float32),
            pltpu.VMEM((_CH, B, D), jnp.float32),
            pltpu.VMEM((_CH, D), jnp.float32),
            pltpu.VMEM((_CH, D), jnp.float32),
            pltpu.VMEM((2, D), jnp.float32),
            pltpu.SemaphoreType.DMA,
            pltpu.SemaphoreType.DMA,
            pltpu.SemaphoreType.DMA,
            pltpu.SemaphoreType.DMA,
        ],
    )
    partials = sc_stats(x, pe)

    sums_tc = pl.pallas_call(
        _tc_stats_body,
        grid=(_S_TC // _SB,),
        in_specs=[
            pl.BlockSpec((_SB, B, D), lambda i: (i, 0, 0)),
            pl.BlockSpec((_SB, D), lambda i: (i, 0)),
        ],
        out_specs=pl.BlockSpec((2, B, D), lambda i: (0, 0, 0)),
        out_shape=jax.ShapeDtypeStruct((2, B, D), jnp.float32),
    )(x, pe)

    go = pl.pallas_call(
        _scale_body,
        out_shape=jax.ShapeDtypeStruct((2, D), jnp.float32),
    )(partials, sums_tc, gamma.reshape(1, D), beta.reshape(1, D))

    out = pl.pallas_call(
        _tc_norm_body,
        grid=(S // _SBN,),
        in_specs=[
            pl.BlockSpec((2, D), lambda i: (0, 0)),
            pl.BlockSpec((_SBN, B, D), lambda i: (i, 0, 0)),
            pl.BlockSpec((_SBN, D), lambda i: (i, 0)),
        ],
        out_specs=pl.BlockSpec((_SBN, B, D), lambda i: (i, 0, 0)),
        out_shape=jax.ShapeDtypeStruct((S, B, D), jnp.float32),
    )(go, x, pe)

    return out
